# baseline (speedup 1.0000x reference)
import functools

import jax
import jax.numpy as jnp
from jax import lax
from jax.experimental import pallas as pl
from jax.experimental.pallas import tpu as pltpu
from jax.experimental.pallas import tpu_sc as plsc

N, D, E = 10000, 128, 320000
NC, NS = 2, 16
NW = NC * NS
CH = 128
NJ = 80
NJR = 40
PADE = NW * NJ * CH
NPAD = 10240
RPT = NPAD // NS


def _sc_agg_body(x_hbm, src_hbm, dst_hbm, zero_hbm, out_hbm,
                 agg_sh, src_v, dst_v, rows0, rows1,
                 gsem0, gsem1, ssem0, ssem1):
    c = lax.axis_index("c")
    s = lax.axis_index("s")
    wid = s * NC + c

    pltpu.sync_copy(zero_hbm, agg_sh.at[pl.ds(s * RPT, RPT)])
    plsc.subcore_barrier()

    ebase = wid * NJ * CH

    def chunk(j, carry):
        base = ebase + j * CH
        pltpu.sync_copy(src_hbm.at[pl.ds(base, CH)], src_v)
        pltpu.sync_copy(dst_hbm.at[pl.ds(base, CH)], dst_v)
        pltpu.async_copy(x_hbm.at[src_v], rows0, gsem0).wait()
        pltpu.sync_copy(rows0, agg_sh.at[dst_v], add=True)
        return carry

    lax.fori_loop(0, NJ, chunk, 0)

    plsc.subcore_barrier()
    pltpu.sync_copy(agg_sh.at[pl.ds(s * RPT, RPT)],
                    out_hbm.at[c, pl.ds(s * RPT, RPT)])


@functools.cache
def _sc_agg():
    return pl.kernel(
        _sc_agg_body,
        mesh=plsc.VectorSubcoreMesh(core_axis_name="c", subcore_axis_name="s"),
        out_type=jax.ShapeDtypeStruct((NC, NPAD, D), jnp.float32),
        scratch_types=[
            pltpu.VMEM_SHARED((NPAD, D), jnp.float32),
            pltpu.VMEM((CH,), jnp.int32),
            pltpu.VMEM((CH,), jnp.int32),
            pltpu.VMEM((CH, D), jnp.float32),
            pltpu.VMEM((CH, D), jnp.float32),
            pltpu.SemaphoreType.DMA,
            pltpu.SemaphoreType.DMA,
            pltpu.SemaphoreType.DMA,
            pltpu.SemaphoreType.DMA,
        ],
    )


def _tc_body(x_ref, p_ref, wt_ref, b_ref, g_ref, bt_ref, eps_ref, o_ref):
    x = x_ref[...]
    agg = p_ref[0, :N] + p_ref[1, :N]
    u = (1.0 + eps_ref[0, 0]) * x + agg
    h = jnp.dot(u, wt_ref[...], preferred_element_type=jnp.float32) + b_ref[...]
    mean = jnp.mean(h, axis=0, keepdims=True)
    d = h - mean
    var = jnp.mean(d * d, axis=0, keepdims=True)
    hn = d * lax.rsqrt(var + 1e-5) * g_ref[...] + bt_ref[...]
    o_ref[...] = jnp.maximum(hn, 0.0) + x


def kernel(x, edge_index, W, b, eps, gamma, beta):
    npad = PADE - E
    src2 = jnp.concatenate([edge_index[0], jnp.zeros((npad,), jnp.int32)])
    pad_dst = N + (jnp.arange(npad, dtype=jnp.int32) % (NPAD - N))
    dst2 = jnp.concatenate([edge_index[1], pad_dst])
    partials = _sc_agg()(x, src2, dst2, jnp.zeros((RPT, D), jnp.float32))
    return pl.pallas_call(
        _tc_body,
        out_shape=jax.ShapeDtypeStruct((N, D), jnp.float32),
    )(x, partials, W.T,
      b.reshape(1, D), gamma.reshape(1, D), beta.reshape(1, D),
      eps.reshape(1, 1))


# speedup vs baseline: 3.6549x; 3.6549x over previous
import functools

import jax
import jax.numpy as jnp
from jax import lax
from jax.experimental import pallas as pl
from jax.experimental.pallas import tpu as pltpu
from jax.experimental.pallas import tpu_sc as plsc

N, D, E = 10000, 128, 320000
NC, NS = 2, 16
NW = NC * NS
EPT = E // NW
CH = 80
NJ = EPT // CH
NPAIR = NJ // 2
NPAD = 10240
RPT = NPAD // NS


def _sc_agg_body(x_hbm, src_hbm, dst_hbm, zero_hbm, out_hbm,
                 agg_sh, src_v, dst_v, rows0, rows1,
                 gsem0, gsem1, ssem0, ssem1):
    c = lax.axis_index("c")
    s = lax.axis_index("s")
    wid = s * NC + c

    pltpu.sync_copy(zero_hbm, agg_sh.at[pl.ds(s * RPT, RPT)])
    pltpu.sync_copy(src_hbm.at[pl.ds(wid * EPT, EPT)], src_v)
    pltpu.sync_copy(dst_hbm.at[pl.ds(wid * EPT, EPT)], dst_v)
    plsc.subcore_barrier()

    def gather(j, buf, sem):
        return pltpu.async_copy(x_hbm.at[src_v.at[pl.ds(j * CH, CH)]],
                                buf, sem)

    def scatter(j, buf, sem):
        return pltpu.async_copy(buf, agg_sh.at[dst_v.at[pl.ds(j * CH, CH)]],
                                sem, add=True)

    gather(0, rows0, gsem0)
    gather(1, rows1, gsem1)

    def pair(jj, carry):
        j0 = 2 * jj
        j1 = j0 + 1
        gather(j0, rows0, gsem0).wait()
        s0 = scatter(j0, rows0, ssem0)
        gather(j1, rows1, gsem1).wait()
        s1 = scatter(j1, rows1, ssem1)
        s0.wait()

        @pl.when(jj < NPAIR - 1)
        def _prefetch0():
            gather(j0 + 2, rows0, gsem0)

        s1.wait()

        @pl.when(jj < NPAIR - 1)
        def _prefetch1():
            gather(j1 + 2, rows1, gsem1)

        return carry

    lax.fori_loop(0, NPAIR, pair, 0)

    gather(NJ - 1, rows0, gsem0).wait()
    scatter(NJ - 1, rows0, ssem0).wait()

    plsc.subcore_barrier()
    pltpu.sync_copy(agg_sh.at[pl.ds(s * RPT, RPT)],
                    out_hbm.at[c, pl.ds(s * RPT, RPT)])


@functools.cache
def _sc_agg():
    return pl.kernel(
        _sc_agg_body,
        mesh=plsc.VectorSubcoreMesh(core_axis_name="c", subcore_axis_name="s"),
        out_type=jax.ShapeDtypeStruct((NC, NPAD, D), jnp.float32),
        scratch_types=[
            pltpu.VMEM_SHARED((NPAD, D), jnp.float32),
            pltpu.VMEM((EPT,), jnp.int32),
            pltpu.VMEM((EPT,), jnp.int32),
            pltpu.VMEM((CH, D), jnp.float32),
            pltpu.VMEM((CH, D), jnp.float32),
            pltpu.SemaphoreType.DMA,
            pltpu.SemaphoreType.DMA,
            pltpu.SemaphoreType.DMA,
            pltpu.SemaphoreType.DMA,
        ],
    )


def _tc_body(x_ref, p_ref, wt_ref, b_ref, g_ref, bt_ref, eps_ref, o_ref):
    x = x_ref[...]
    agg = p_ref[0, :N] + p_ref[1, :N]
    u = (1.0 + eps_ref[0, 0]) * x + agg
    h = jnp.dot(u, wt_ref[...], preferred_element_type=jnp.float32) + b_ref[...]
    mean = jnp.mean(h, axis=0, keepdims=True)
    d = h - mean
    var = jnp.mean(d * d, axis=0, keepdims=True)
    hn = d * lax.rsqrt(var + 1e-5) * g_ref[...] + bt_ref[...]
    o_ref[...] = jnp.maximum(hn, 0.0) + x


def kernel(x, edge_index, W, b, eps, gamma, beta):
    partials = _sc_agg()(x, edge_index[0], edge_index[1],
                         jnp.zeros((RPT, D), jnp.float32))
    return pl.pallas_call(
        _tc_body,
        out_shape=jax.ShapeDtypeStruct((N, D), jnp.float32),
    )(x, partials, W.T,
      b.reshape(1, D), gamma.reshape(1, D), beta.reshape(1, D),
      eps.reshape(1, 1))
